# baseline (speedup 1.0000x reference)
import jax
import jax.numpy as jnp
from jax.experimental import pallas as pl
from jax.experimental.pallas import tpu as pltpu

_IMAGE_WIDTH = 640
_IMAGE_HEIGHT = 480
_MAX_PIXEL = _IMAGE_WIDTH * _IMAGE_HEIGHT - 1
_NSAMP = 256
_BLOCK_ROWS = 5120


def _ring_sampler(matches, neg_table, indices):
    n = matches.shape[0]
    num_off = neg_table.shape[0]
    n_blocks = -(-n // _BLOCK_ROWS)
    matches = jnp.pad(matches, (0, n_blocks * _BLOCK_ROWS - n))

    def body(neg_ref, idx_ref, m_ref, out_ref):
        idx_v = idx_ref[...]
        off = jnp.zeros((_NSAMP,), jnp.int32)
        for t in range(num_off):
            off = jnp.where(idx_v == t, neg_ref[t], off)
        m = m_ref[...]
        out_ref[...] = jnp.minimum(
            jnp.maximum(m[:, None] + off[None, :], 0), _MAX_PIXEL)

    return pl.pallas_call(
        body,
        grid=(n_blocks,),
        in_specs=[
            pl.BlockSpec(memory_space=pltpu.SMEM),
            pl.BlockSpec((_NSAMP,), lambda i: (0,)),
            pl.BlockSpec((_BLOCK_ROWS,), lambda i: (i,)),
        ],
        out_specs=pl.BlockSpec((_BLOCK_ROWS, _NSAMP), lambda i: (i, 0)),
        out_shape=jax.ShapeDtypeStruct((n, _NSAMP), jnp.int32),
    )(neg_table, indices, matches)


def kernel(num_samples, matches_b, negative_offsets):
    del num_samples
    num_off = negative_offsets.shape[0]

    key = jax.random.fold_in(jax.random.key(0), 1)
    indices = jax.random.randint(key, (_NSAMP,), 0, num_off, dtype=jnp.int32)

    return _ring_sampler(matches_b, negative_offsets, indices)


# speedup vs baseline: 1.0082x; 1.0082x over previous
import jax
import jax.numpy as jnp
from jax.experimental import pallas as pl
from jax.experimental.pallas import tpu as pltpu

_IMAGE_WIDTH = 640
_IMAGE_HEIGHT = 480
_MAX_PIXEL = _IMAGE_WIDTH * _IMAGE_HEIGHT - 1
_NSAMP = 256
_BLOCK_ROWS = 7168


def _ring_sampler(matches, neg_table, indices):
    n = matches.shape[0]
    num_off = neg_table.shape[0]
    n_blocks = -(-n // _BLOCK_ROWS)
    matches = jnp.pad(matches, (0, n_blocks * _BLOCK_ROWS - n))

    def body(neg_ref, idx_ref, m_ref, out_ref):
        idx_v = idx_ref[...]
        off = jnp.zeros((_NSAMP,), jnp.int32)
        for t in range(num_off):
            off = jnp.where(idx_v == t, neg_ref[t], off)
        m = m_ref[...]
        out_ref[...] = jnp.minimum(
            jnp.maximum(m[:, None] + off[None, :], 0), _MAX_PIXEL)

    return pl.pallas_call(
        body,
        grid=(n_blocks,),
        in_specs=[
            pl.BlockSpec(memory_space=pltpu.SMEM),
            pl.BlockSpec((_NSAMP,), lambda i: (0,)),
            pl.BlockSpec((_BLOCK_ROWS,), lambda i: (i,)),
        ],
        out_specs=pl.BlockSpec((_BLOCK_ROWS, _NSAMP), lambda i: (i, 0)),
        out_shape=jax.ShapeDtypeStruct((n, _NSAMP), jnp.int32),
    )(neg_table, indices, matches)


def kernel(num_samples, matches_b, negative_offsets):
    del num_samples
    num_off = negative_offsets.shape[0]

    key = jax.random.fold_in(jax.random.key(0), 1)
    indices = jax.random.randint(key, (_NSAMP,), 0, num_off, dtype=jnp.int32)

    return _ring_sampler(matches_b, negative_offsets, indices)
